# baseline (speedup 1.0000x reference)
from functools import partial

import jax
import jax.numpy as jnp
from jax.experimental import pallas as pl
from jax.experimental.pallas import tpu as pltpu

_CHUNKS = 8


def _eca_fused_kernel(x_hbm, w_ref, o_hbm, x_buf, in_sem, out_sem, *, k, inv_hw,
                      base=0):
    j = pl.program_id(1)
    nb = pl.num_programs(1)
    b = pl.program_id(0) * nb + j
    slot = jax.lax.rem(j, 2)
    nxt = jax.lax.rem(j + 1, 2)
    c = x_buf.shape[1]
    cc = c // _CHUNKS
    pad = (k - 1) // 2

    def in_copy(buf_slot, batch, q):
        return pltpu.make_async_copy(
            x_hbm.at[pl.ds((base + batch) * c + q * cc, cc)],
            x_buf.at[buf_slot, pl.ds(q * cc, cc)],
            in_sem.at[buf_slot, q])

    def out_copy(buf_slot, batch, q):
        return pltpu.make_async_copy(
            x_buf.at[buf_slot, pl.ds(q * cc, cc)],
            o_hbm.at[pl.ds(batch * c + q * cc, cc)],
            out_sem.at[buf_slot, q])

    @pl.when(j == 0)
    def _():
        for q in range(_CHUNKS):
            in_copy(slot, b, q).start()

    @pl.when(j + 1 < nb)
    def _():
        @pl.when(j >= 1)
        def _():
            for q in range(_CHUNKS):
                out_copy(nxt, b - 1, q).wait()
        for q in range(_CHUNKS):
            in_copy(nxt, b + 1, q).start()

    parts = []
    for q in range(_CHUNKS):
        in_copy(slot, b, q).wait()
        xq = x_buf[slot, q * cc:(q + 1) * cc]
        parts.append(jnp.sum(xq, axis=(1, 2), keepdims=True))
    s = jnp.concatenate(parts, axis=0)

    zpad = jnp.zeros((pad, 1, 1), jnp.float32)
    sp = jnp.concatenate([zpad, s, zpad], axis=0)
    y = w_ref[0] * sp[0:c]
    for t in range(1, k):
        y = y + w_ref[t] * sp[t:t + c]
    gate = jax.nn.sigmoid(y * inv_hw)

    for q in range(_CHUNKS):
        x_buf[slot, q * cc:(q + 1) * cc] = (
            x_buf[slot, q * cc:(q + 1) * cc] * gate[q * cc:(q + 1) * cc])
        out_copy(slot, b, q).start()

    @pl.when(j == nb - 1)
    def _():
        @pl.when(j >= 1)
        def _():
            for q in range(_CHUNKS):
                out_copy(nxt, b - 1, q).wait()
        for q in range(_CHUNKS):
            out_copy(slot, b, q).wait()


def _eca_part(x3, w32, n_part, c, h, w, k, base):
    cores = 2 if n_part % 2 == 0 else 1
    return pl.pallas_call(
        partial(_eca_fused_kernel, k=k, inv_hw=1.0 / (h * w), base=base),
        out_shape=jax.ShapeDtypeStruct((n_part * c, h, w), x3.dtype),
        grid=(cores, n_part // cores),
        in_specs=[
            pl.BlockSpec(memory_space=pltpu.MemorySpace.HBM),
            pl.BlockSpec(memory_space=pltpu.MemorySpace.SMEM),
        ],
        out_specs=pl.BlockSpec(memory_space=pltpu.MemorySpace.HBM),
        scratch_shapes=[
            pltpu.VMEM((2, c, h, w), jnp.float32),
            pltpu.SemaphoreType.DMA((2, _CHUNKS)),
            pltpu.SemaphoreType.DMA((2, _CHUNKS)),
        ],
        compiler_params=pltpu.CompilerParams(
            dimension_semantics=("parallel", "arbitrary"),
            vmem_limit_bytes=56 * 1024 * 1024,
        ),
    )(x3, w32)


def kernel(x_nchw, w_taps):
    n, c, h, w = x_nchw.shape
    k = w_taps.shape[0]
    w32 = w_taps.astype(jnp.float32)

    x3 = x_nchw.reshape(n * c, h, w)
    if n % 2 == 0:
        hn = n // 2
        o0 = _eca_part(x3, w32, hn, c, h, w, k, 0)
        o1 = _eca_part(x3, w32, hn, c, h, w, k, hn)
        out3 = jnp.concatenate([o0, o1], axis=0)
    else:
        out3 = _eca_part(x3, w32, n, c, h, w, k, 0)

    return out3.reshape(n, c, h, w)


# speedup vs baseline: 1.4305x; 1.4305x over previous
from functools import partial

import jax
import jax.numpy as jnp
from jax.experimental import pallas as pl
from jax.experimental.pallas import tpu as pltpu

_CHUNKS = 8


def _eca_fused_kernel(x_hbm, w_ref, o_hbm, x_buf, in_sem, out_sem, *, k, inv_hw):
    j = pl.program_id(1)
    nb = pl.num_programs(1)
    b = pl.program_id(0) * nb + j
    slot = jax.lax.rem(j, 2)
    nxt = jax.lax.rem(j + 1, 2)
    c = x_buf.shape[1]
    cc = c // _CHUNKS
    pad = (k - 1) // 2

    def in_copy(buf_slot, batch, q):
        return pltpu.make_async_copy(
            x_hbm.at[pl.ds(batch * c + q * cc, cc)],
            x_buf.at[buf_slot, pl.ds(q * cc, cc)],
            in_sem.at[buf_slot, q])

    def out_copy(buf_slot, batch, q):
        return pltpu.make_async_copy(
            x_buf.at[buf_slot, pl.ds(q * cc, cc)],
            o_hbm.at[pl.ds(batch * c + q * cc, cc)],
            out_sem.at[buf_slot, q])

    @pl.when(j == 0)
    def _():
        for q in range(_CHUNKS):
            in_copy(slot, b, q).start()

    @pl.when(j + 1 < nb)
    def _():
        @pl.when(j >= 1)
        def _():
            for q in range(_CHUNKS):
                out_copy(nxt, b - 1, q).wait()
        for q in range(_CHUNKS):
            in_copy(nxt, b + 1, q).start()

    parts = []
    for q in range(_CHUNKS):
        in_copy(slot, b, q).wait()
        xq = x_buf[slot, q * cc:(q + 1) * cc]
        parts.append(jnp.sum(xq, axis=(1, 2), keepdims=True))
    s = jnp.concatenate(parts, axis=0)

    zpad = jnp.zeros((pad, 1, 1), jnp.float32)
    sp = jnp.concatenate([zpad, s, zpad], axis=0)
    y = w_ref[0] * sp[0:c]
    for t in range(1, k):
        y = y + w_ref[t] * sp[t:t + c]
    gate = jax.nn.sigmoid(y * inv_hw)

    for q in range(_CHUNKS):
        x_buf[slot, q * cc:(q + 1) * cc] = (
            x_buf[slot, q * cc:(q + 1) * cc] * gate[q * cc:(q + 1) * cc])
        out_copy(slot, b, q).start()

    @pl.when(j == nb - 1)
    def _():
        @pl.when(j >= 1)
        def _():
            for q in range(_CHUNKS):
                out_copy(nxt, b - 1, q).wait()
        for q in range(_CHUNKS):
            out_copy(slot, b, q).wait()


def kernel(x_nchw, w_taps):
    n, c, h, w = x_nchw.shape
    k = w_taps.shape[0]
    cores = 2 if n % 2 == 0 else 1

    x3 = x_nchw.reshape(n * c, h, w)
    out3 = pl.pallas_call(
        partial(_eca_fused_kernel, k=k, inv_hw=1.0 / (h * w)),
        out_shape=jax.ShapeDtypeStruct((n * c, h, w), x_nchw.dtype),
        grid=(cores, n // cores),
        in_specs=[
            pl.BlockSpec(memory_space=pltpu.MemorySpace.HBM),
            pl.BlockSpec(memory_space=pltpu.MemorySpace.SMEM),
        ],
        out_specs=pl.BlockSpec(memory_space=pltpu.MemorySpace.HBM),
        scratch_shapes=[
            pltpu.VMEM((2, c, h, w), jnp.float32),
            pltpu.SemaphoreType.DMA((2, _CHUNKS)),
            pltpu.SemaphoreType.DMA((2, _CHUNKS)),
        ],
        compiler_params=pltpu.CompilerParams(
            dimension_semantics=("parallel", "arbitrary"),
            vmem_limit_bytes=56 * 1024 * 1024,
        ),
    )(x3, w_taps.astype(jnp.float32))

    return out3.reshape(n, c, h, w)


# speedup vs baseline: 1.4305x; 1.0000x over previous
from functools import partial

import jax
import jax.numpy as jnp
from jax.experimental import pallas as pl
from jax.experimental.pallas import tpu as pltpu

_CHUNKS = 16


def _eca_fused_kernel(x_hbm, w_ref, o_hbm, x_buf, in_sem, out_sem, *, k, inv_hw):
    j = pl.program_id(1)
    nb = pl.num_programs(1)
    b = pl.program_id(0) * nb + j
    slot = jax.lax.rem(j, 2)
    nxt = jax.lax.rem(j + 1, 2)
    c = x_buf.shape[1]
    cc = c // _CHUNKS
    pad = (k - 1) // 2

    def in_copy(buf_slot, batch, q):
        return pltpu.make_async_copy(
            x_hbm.at[pl.ds(batch * c + q * cc, cc)],
            x_buf.at[buf_slot, pl.ds(q * cc, cc)],
            in_sem.at[buf_slot, q])

    def out_copy(buf_slot, batch, q):
        return pltpu.make_async_copy(
            x_buf.at[buf_slot, pl.ds(q * cc, cc)],
            o_hbm.at[pl.ds(batch * c + q * cc, cc)],
            out_sem.at[buf_slot, q])

    @pl.when(j == 0)
    def _():
        for q in range(_CHUNKS):
            in_copy(slot, b, q).start()

    @pl.when(j + 1 < nb)
    def _():
        @pl.when(j >= 1)
        def _():
            for q in range(_CHUNKS):
                out_copy(nxt, b - 1, q).wait()
        for q in range(_CHUNKS):
            in_copy(nxt, b + 1, q).start()

    parts = []
    for q in range(_CHUNKS):
        in_copy(slot, b, q).wait()
        xq = x_buf[slot, q * cc:(q + 1) * cc]
        parts.append(jnp.sum(xq, axis=(1, 2), keepdims=True))
    s = jnp.concatenate(parts, axis=0)

    zpad = jnp.zeros((pad, 1, 1), jnp.float32)
    sp = jnp.concatenate([zpad, s, zpad], axis=0)
    y = w_ref[0] * sp[0:c]
    for t in range(1, k):
        y = y + w_ref[t] * sp[t:t + c]
    gate = jax.nn.sigmoid(y * inv_hw)

    for q in range(_CHUNKS):
        x_buf[slot, q * cc:(q + 1) * cc] = (
            x_buf[slot, q * cc:(q + 1) * cc] * gate[q * cc:(q + 1) * cc])
        out_copy(slot, b, q).start()

    @pl.when(j == nb - 1)
    def _():
        @pl.when(j >= 1)
        def _():
            for q in range(_CHUNKS):
                out_copy(nxt, b - 1, q).wait()
        for q in range(_CHUNKS):
            out_copy(slot, b, q).wait()


def kernel(x_nchw, w_taps):
    n, c, h, w = x_nchw.shape
    k = w_taps.shape[0]
    cores = 2 if n % 2 == 0 else 1

    x3 = x_nchw.reshape(n * c, h, w)
    out3 = pl.pallas_call(
        partial(_eca_fused_kernel, k=k, inv_hw=1.0 / (h * w)),
        out_shape=jax.ShapeDtypeStruct((n * c, h, w), x_nchw.dtype),
        grid=(cores, n // cores),
        in_specs=[
            pl.BlockSpec(memory_space=pltpu.MemorySpace.HBM),
            pl.BlockSpec(memory_space=pltpu.MemorySpace.SMEM),
        ],
        out_specs=pl.BlockSpec(memory_space=pltpu.MemorySpace.HBM),
        scratch_shapes=[
            pltpu.VMEM((2, c, h, w), jnp.float32),
            pltpu.SemaphoreType.DMA((2, _CHUNKS)),
            pltpu.SemaphoreType.DMA((2, _CHUNKS)),
        ],
        compiler_params=pltpu.CompilerParams(
            dimension_semantics=("parallel", "arbitrary"),
            vmem_limit_bytes=56 * 1024 * 1024,
        ),
    )(x3, w_taps.astype(jnp.float32))

    return out3.reshape(n, c, h, w)
